# baseline (speedup 1.0000x reference)
import functools

import jax
import jax.numpy as jnp
from jax import lax
from jax.experimental import pallas as pl
from jax.experimental.pallas import tpu as pltpu
from jax.experimental.pallas import tpu_sc as plsc

T = 16384
D = 2048
E = 64
G = 8
EPG = 8
S = 4
K = 8

BLK = 2048
NBLK = T // BLK

NC = 2
NS = 16
L = 16
NW = NC * NS
TPW = T // NW
SUB = 128
NSUB = TPW // SUB
CH = SUB // L
NCH = T // SUB


def _logits_kernel(x_ref, w_ref, lt_ref):
    lt = jax.lax.dot_general(
        w_ref[...], x_ref[...],
        dimension_numbers=(((1,), (1,)), ((), ())),
        preferred_element_type=jnp.float32,
    )
    for s2 in range(BLK // SUB):
        lt_ref[s2] = lt[:, s2 * SUB:(s2 + 1) * SUB]


def _aux_kernel(ps_ref, hs_ref, aux_ref):
    aux_ref[...] = (jnp.sum(jnp.sum(ps_ref[...], axis=0, keepdims=True)
                            * jnp.sum(hs_ref[...], axis=0, keepdims=True))
                    * (float(E) / (float(T) * float(T)))).reshape(1, 1)


def _sc_route(lt_hbm, w_hbm, id_hbm, ps_hbm, hs_hbm,
              ltb, wb, idb, psb, hsb, psbT, hsbT):
    wid = lax.axis_index("s") * NC + lax.axis_index("c")
    base = wid * TPW

    lanes = lax.iota(jnp.int32, L)
    ones = jnp.ones((L,), jnp.float32)
    bc = jax.lax.bitcast_convert_type

    for e in range(E):
        psb[pl.ds(e * L, L)] = jnp.zeros((L,), jnp.float32)
        hsb[pl.ds(e * L, L)] = jnp.zeros((L,), jnp.float32)

    for b in range(NSUB):
        pltpu.sync_copy(lt_hbm.at[pl.ds((wid * NSUB + b) * (E * SUB), E * SUB)], ltb)

        def chunk(c, _):
            off = c * L
            m = ltb[pl.ds(off, L)]
            for e in range(1, E):
                m = jnp.maximum(m, ltb[pl.ds(e * SUB + off, L)])
            z = jnp.zeros((L,), jnp.float32)
            exs = []
            for e in range(E):
                ex = jnp.exp(ltb[pl.ds(e * SUB + off, L)] - m)
                exs.append(ex)
                z = z + ex
            inv = 1.0 / z
            k1s, k2s, gks = [], [], []
            for g in range(G):
                k1 = jnp.full((L,), -1.0, jnp.float32)
                k2 = jnp.full((L,), -1.0, jnp.float32)
                for j in range(EPG):
                    e = g * EPG + j
                    p = exs[e] * inv
                    psb[pl.ds(e * L, L)] = psb[pl.ds(e * L, L)] + p
                    kf = bc((bc(p, jnp.int32) & -64) | (63 - e), jnp.float32)
                    gt = kf > k1
                    k2 = jnp.where(gt, k1, jnp.maximum(k2, kf))
                    k1 = jnp.where(gt, kf, k1)
                gs = (bc(bc(k1, jnp.int32) & -64, jnp.float32)
                      + bc(bc(k2, jnp.int32) & -64, jnp.float32))
                gks.append(bc((bc(gs, jnp.int32) & -8) | (7 - g), jnp.float32))
                k1s.append(k1)
                k2s.append(k2)
            ranks = []
            for g in range(G):
                r = jnp.zeros((L,), jnp.int32)
                for h in range(G):
                    r = r + (gks[g] < gks[h]).astype(jnp.int32)
                ranks.append(r)
            vals, ids = [], []
            denom = jnp.full((L,), 1e-9, jnp.float32)
            for s in range(S):
                k1_s = jnp.zeros((L,), jnp.float32)
                k2_s = jnp.zeros((L,), jnp.float32)
                for g in range(G):
                    onsel = ranks[g] == s
                    k1_s = jnp.where(onsel, k1s[g], k1_s)
                    k2_s = jnp.where(onsel, k2s[g], k2_s)
                b1 = bc(k1_s, jnp.int32)
                b2 = bc(k2_s, jnp.int32)
                v1 = bc(b1 & -64, jnp.float32)
                v2 = bc(b2 & -64, jnp.float32)
                vals.extend([v1, v2])
                ids.extend([63 - (b1 & 63), 63 - (b2 & 63)])
                denom = denom + v1 + v2
            winv = 1.0 / denom
            rows = (b * SUB) + off + lanes
            for j in range(K):
                plsc.store_scatter(wb, [rows * K + j], vals[j] * winv)
                plsc.store_scatter(idb, [rows * K + j], ids[j])
            plsc.addupdate_scatter(hsb, [ids[0] * L + lanes], ones)

        lax.fori_loop(0, CH, chunk, None)

    for e in range(E):
        plsc.store_scatter(psbT, [lanes * E + e], psb[pl.ds(e * L, L)])
        plsc.store_scatter(hsbT, [lanes * E + e], hsb[pl.ds(e * L, L)])

    pltpu.sync_copy(wb, w_hbm.at[pl.ds(base * K, TPW * K)])
    pltpu.sync_copy(idb, id_hbm.at[pl.ds(base * K, TPW * K)])
    pltpu.sync_copy(psbT, ps_hbm.at[pl.ds(wid * L * E, L * E)])
    pltpu.sync_copy(hsbT, hs_hbm.at[pl.ds(wid * L * E, L * E)])


@jax.jit
def kernel(x, W):
    lt = pl.pallas_call(
        _logits_kernel,
        grid=(NBLK,),
        in_specs=[
            pl.BlockSpec((BLK, D), lambda i: (i, 0)),
            pl.BlockSpec((E, D), lambda i: (0, 0)),
        ],
        out_specs=pl.BlockSpec((BLK // SUB, E, SUB), lambda i: (i, 0, 0)),
        out_shape=jax.ShapeDtypeStruct((NCH, E, SUB), jnp.float32),
    )(x, W)
    lt1 = lt.reshape(NCH * E * SUB)

    route = functools.partial(
        pl.kernel,
        mesh=plsc.VectorSubcoreMesh(core_axis_name="c", subcore_axis_name="s"),
        compiler_params=pltpu.CompilerParams(needs_layout_passes=False),
        out_type=[
            jax.ShapeDtypeStruct((T * K,), jnp.float32),
            jax.ShapeDtypeStruct((T * K,), jnp.int32),
            jax.ShapeDtypeStruct((NW * L * E,), jnp.float32),
            jax.ShapeDtypeStruct((NW * L * E,), jnp.float32),
        ],
        scratch_types=[
            pltpu.VMEM((E * SUB,), jnp.float32),
            pltpu.VMEM((TPW * K,), jnp.float32),
            pltpu.VMEM((TPW * K,), jnp.int32),
            pltpu.VMEM((E * L,), jnp.float32),
            pltpu.VMEM((E * L,), jnp.float32),
            pltpu.VMEM((L * E,), jnp.float32),
            pltpu.VMEM((L * E,), jnp.float32),
        ],
    )(_sc_route)
    w_out, id_out, ps, hs = route(lt1)

    aux = pl.pallas_call(
        _aux_kernel,
        out_shape=jax.ShapeDtypeStruct((1, 1), jnp.float32),
    )(ps.reshape(NW * L, E), hs.reshape(NW * L, E))
    return w_out.reshape(T, K), id_out.reshape(T, K), aux.reshape(())


# speedup vs baseline: 1.8190x; 1.8190x over previous
import functools

import jax
import jax.numpy as jnp
from jax.experimental import pallas as pl
from jax.experimental.pallas import tpu as pltpu

T = 16384
D = 2048
E = 64
G = 8
EPG = 8
TKG = 2
S = 4
K = 8

BLK = 2048
NBLK = T // BLK


def _router_kernel(x_ref, wt_ref, w_ref, id_ref, aux_ref, hist_ref, psum_ref):
    i = pl.program_id(0)

    lt = jax.lax.dot_general(
        wt_ref[...], x_ref[...],
        dimension_numbers=(((1,), (1,)), ((), ())),
        preferred_element_type=jnp.float32,
    )

    m = jnp.max(lt, axis=0, keepdims=True)
    ex = jnp.exp(lt - m)
    z = jnp.sum(ex, axis=0, keepdims=True)
    p = ex * (1.0 / z)

    bc = jax.lax.bitcast_convert_type

    iota64 = jax.lax.broadcasted_iota(jnp.int32, (E, BLK), 0)
    keys = bc((bc(p, jnp.int32) & -64) | (63 - iota64), jnp.float32)

    k1r, k2r, gkr = [], [], []
    for g in range(G):
        kg = keys[g * EPG:(g + 1) * EPG, :]
        k1 = jnp.max(kg, axis=0, keepdims=True)
        k2 = jnp.max(jnp.where(kg == k1, -1.0, kg), axis=0, keepdims=True)
        gs = bc(bc(k1, jnp.int32) & -64, jnp.float32) + bc(bc(k2, jnp.int32) & -64, jnp.float32)
        gkr.append(bc((bc(gs, jnp.int32) & -8) | (7 - g), jnp.float32))
        k1r.append(k1)
        k2r.append(k2)
    k1s = jnp.concatenate(k1r, axis=0)
    k2s = jnp.concatenate(k2r, axis=0)
    gk = jnp.concatenate(gkr, axis=0)

    rank = jnp.zeros((G, BLK), jnp.int32)
    for h in range(G):
        rank = rank + (gk < gk[h:h + 1, :]).astype(jnp.int32)

    rows_w, rows_id, denom = [], [], jnp.float32(1e-9)
    for s in range(S):
        onsel = rank == s
        b1 = bc(jnp.sum(jnp.where(onsel, k1s, 0.0), axis=0, keepdims=True), jnp.int32)
        b2 = bc(jnp.sum(jnp.where(onsel, k2s, 0.0), axis=0, keepdims=True), jnp.int32)
        v1 = bc(b1 & -64, jnp.float32)
        v2 = bc(b2 & -64, jnp.float32)
        rows_w.extend([v1, v2])
        rows_id.extend([63 - (b1 & 63), 63 - (b2 & 63)])
        denom = denom + v1 + v2

    wt_t = jnp.concatenate(rows_w, axis=0) / denom
    ids_t = jnp.concatenate(rows_id, axis=0)

    w_ref[...] = wt_t.T
    id_ref[...] = ids_t.T

    top1 = ids_t[0:1, :]
    one_hot = (jax.lax.broadcasted_iota(jnp.int32, (E, BLK), 0) == top1).astype(jnp.float32)

    @pl.when(i == 0)
    def _init():
        hist_ref[...] = jnp.zeros_like(hist_ref)
        psum_ref[...] = jnp.zeros_like(psum_ref)

    hist_ref[...] += jnp.sum(one_hot, axis=1, keepdims=True)
    psum_ref[...] += jnp.sum(p, axis=1, keepdims=True)

    @pl.when(i == NBLK - 1)
    def _fin():
        aux_ref[...] = (jnp.sum(hist_ref[...] * psum_ref[...])
                        * (float(E) / (float(T) * float(T)))).reshape(1, 1)


@jax.jit
def kernel(x, W):
    w_out, id_out, aux = pl.pallas_call(
        _router_kernel,
        grid=(NBLK,),
        in_specs=[
            pl.BlockSpec((BLK, D), lambda i: (i, 0)),
            pl.BlockSpec((E, D), lambda i: (0, 0)),
        ],
        out_specs=[
            pl.BlockSpec((BLK, K), lambda i: (i, 0)),
            pl.BlockSpec((BLK, K), lambda i: (i, 0)),
            pl.BlockSpec((1, 1), lambda i: (0, 0)),
        ],
        out_shape=[
            jax.ShapeDtypeStruct((T, K), jnp.float32),
            jax.ShapeDtypeStruct((T, K), jnp.int32),
            jax.ShapeDtypeStruct((1, 1), jnp.float32),
        ],
        scratch_shapes=[
            pltpu.VMEM((E, 1), jnp.float32),
            pltpu.VMEM((E, 1), jnp.float32),
        ],
    )(x, W)
    return w_out, id_out, aux.reshape(())
